# baseline (speedup 1.0000x reference)
import functools

import jax
import jax.numpy as jnp
from jax import lax
from jax.experimental import pallas as pl
from jax.experimental.pallas import tpu as pltpu
from jax.experimental.pallas import tpu_sc as plsc

N = 10000
H = 128
V = 3
E = 320000
P = 50000
ALPHA = 0.5

NPAD = 10240
NC = 2
NS = 16
NW = NC * NS
K = 80
EPADV = E
EPT = EPADV // NW
NCHUNK = EPT // K
ROWS_PT = NPAD // NS
TP = 307200
PPT = TP // NW
PCHUNK = PPT // K

def _elu(x):
    return jnp.where(x > 0, x, jnp.exp(jnp.minimum(x, 0.0)) - 1.0)


def _mesh():
    return plsc.VectorSubcoreMesh(
        core_axis_name="c", subcore_axis_name="s",
        num_cores=NC, num_subcores=NS)


def _deg_body(dst_hbm, out_hbm, idx0_v, idx1_v, ones_v, zeros_v, stg_v, cmp_v,
              acc_sh, si0, si1, ss0, ss1):
    cid = lax.axis_index("c")
    sid = lax.axis_index("s")
    wid = cid * NS + sid
    ebase = wid * EPT
    rbase = sid * ROWS_PT

    def init_ones(r, carry):
        ones_v[r, :] = jnp.ones((16,), jnp.float32)
        return carry

    lax.fori_loop(0, K, init_ones, 0)

    def init_zeros(r, carry):
        zeros_v[r, :] = jnp.zeros((16,), jnp.float32)
        return carry

    lax.fori_loop(0, ROWS_PT, init_zeros, 0)

    for i in range(V):
        pltpu.sync_copy(zeros_v, acc_sh.at[pl.ds(rbase, ROWS_PT)])
        plsc.subcore_barrier()
        base = i * EPADV + ebase

        pltpu.async_copy(dst_hbm.at[pl.ds(pl.multiple_of(base, 8), K)],
                         idx0_v, si0)

        def piter(g, carry):
            c0 = 2 * g

            @pl.when(g > 0)
            def _():
                pltpu.make_async_copy(ones_v, acc_sh.at[idx1_v], ss1).wait()
            d_i1 = pltpu.async_copy(
                dst_hbm.at[pl.ds(pl.multiple_of(base + (c0 + 1) * K, 8), K)],
                idx1_v, si1)
            pltpu.make_async_copy(dst_hbm, idx0_v, si0).wait()
            d_s0 = pltpu.async_copy(ones_v, acc_sh.at[idx0_v], ss0, add=True)
            d_i1.wait()
            pltpu.async_copy(ones_v, acc_sh.at[idx1_v], ss1, add=True)
            d_s0.wait()
            pltpu.async_copy(
                dst_hbm.at[pl.ds(pl.multiple_of(base + (c0 + 2) * K, 8), K)],
                idx0_v, si0)
            return carry

        lax.fori_loop(0, (NCHUNK - 1) // 2, piter, 0)
        pltpu.make_async_copy(ones_v, acc_sh.at[idx1_v], ss1).wait()
        pltpu.make_async_copy(dst_hbm, idx0_v, si0).wait()
        pltpu.async_copy(ones_v, acc_sh.at[idx0_v], ss0, add=True)
        pltpu.make_async_copy(ones_v, acc_sh.at[idx0_v], ss0).wait()

        plsc.subcore_barrier()
        pltpu.sync_copy(acc_sh.at[pl.ds(rbase, ROWS_PT)], stg_v)
        czero = jnp.zeros((16,), jnp.int32)
        lane16 = lax.iota(jnp.int32, 16)

        def compact(r, carry):
            cnt16 = plsc.load_gather(stg_v, [r * 16 + lane16, czero])
            cmp_v[pl.ds(r * 16, 16)] = cnt16
            return carry

        lax.fori_loop(0, ROWS_PT // 16, compact, 0)
        pltpu.sync_copy(
            cmp_v, out_hbm.at[pl.ds((cid * V + i) * NPAD + rbase, ROWS_PT)])
        plsc.subcore_barrier()


def _msg_body(src_hbm, dst_hbm, yw_hbm, out_hbm, gidx0, didx0, gidx1, didx1,
              rows0, rows1, zrows_v, acc_sh, si0, si1, sg0, sg1, ss0, ss1):
    cid = lax.axis_index("c")
    sid = lax.axis_index("s")
    wid = cid * NS + sid
    rbase = sid * ROWS_PT

    def init_zrows(r, carry):
        for cc in range(H // 32):
            zrows_v[r, pl.ds(cc * 32, 32)] = jnp.zeros((32,), jnp.bfloat16)
        return carry

    lax.fori_loop(0, K, init_zrows, 0)

    for i in range(V):
        base = i * EPADV + wid * EPT

        def cp_idx(c, g_v, d_v, sem):
            off = pl.multiple_of(base + c * K, 8)
            pltpu.async_copy(src_hbm.at[pl.ds(off, K)], g_v, sem)
            pltpu.async_copy(dst_hbm.at[pl.ds(off, K)], d_v, sem)

        def wait_idx(g_v, d_v, sem):
            pltpu.make_async_copy(src_hbm, g_v, sem).wait()
            pltpu.make_async_copy(dst_hbm, d_v, sem).wait()

        for kk in range(ROWS_PT // K):
            pltpu.sync_copy(zrows_v, acc_sh.at[pl.ds(rbase + kk * K, K)])
        plsc.subcore_barrier()

        cp_idx(0, gidx0, didx0, si0)
        wait_idx(gidx0, didx0, si0)
        pltpu.async_copy(yw_hbm.at[gidx0], rows0, sg0)

        def piter(g, carry):
            c0 = 2 * g
            @pl.when(g > 0)
            def _():
                pltpu.make_async_copy(rows1, acc_sh.at[didx1], ss1).wait()
            cp_idx(c0 + 1, gidx1, didx1, si1)
            pltpu.make_async_copy(yw_hbm.at[gidx0], rows0, sg0).wait()
            d_s0 = pltpu.async_copy(rows0, acc_sh.at[didx0], ss0, add=True)
            wait_idx(gidx1, didx1, si1)
            pltpu.async_copy(yw_hbm.at[gidx1], rows1, sg1)
            d_s0.wait()
            cp_idx(c0 + 2, gidx0, didx0, si0)
            wait_idx(gidx0, didx0, si0)
            pltpu.async_copy(yw_hbm.at[gidx0], rows0, sg0)
            pltpu.make_async_copy(yw_hbm.at[gidx1], rows1, sg1).wait()
            pltpu.async_copy(rows1, acc_sh.at[didx1], ss1, add=True)
            return carry

        lax.fori_loop(0, (NCHUNK - 1) // 2, piter, 0)
        pltpu.make_async_copy(rows1, acc_sh.at[didx1], ss1).wait()
        pltpu.make_async_copy(yw_hbm.at[gidx0], rows0, sg0).wait()
        pltpu.async_copy(rows0, acc_sh.at[didx0], ss0, add=True)
        pltpu.make_async_copy(rows0, acc_sh.at[didx0], ss0).wait()

        plsc.subcore_barrier()
        for kk in range(ROWS_PT // K):
            pltpu.sync_copy(acc_sh.at[pl.ds(rbase + kk * K, K)], rows0)
            pltpu.sync_copy(rows0, out_hbm.at[cid, i, pl.ds(rbase + kk * K, K)])
        plsc.subcore_barrier()


def _link_body(xf_hbm, ia_hbm, ib_hbm, out_hbm, ia0, ib0, ia1, ib1, ia2, ib2,
               bufa0, bufb0, bufa1, bufb1, bufa2, bufb2, dots0, si0, si1, si2,
               sga0, sgb0, sga1, sgb1, sga2, sgb2):
    cid = lax.axis_index("c")
    sid = lax.axis_index("s")
    wid = cid * NS + sid
    pbase = wid * PPT
    lane = lax.iota(jnp.int32, 16)

    def cp_idx(c, a_v, b_v, sem):
        off = pl.multiple_of(pbase + c * K, 8)
        pltpu.async_copy(ia_hbm.at[pl.ds(off, K)], a_v, sem)
        pltpu.async_copy(ib_hbm.at[pl.ds(off, K)], b_v, sem)

    def wait_idx(a_v, b_v, sem):
        pltpu.make_async_copy(ia_hbm, a_v, sem).wait()
        pltpu.make_async_copy(ib_hbm, b_v, sem).wait()

    def dots(bufa_v, bufb_v, dots_v):
        def group(g, carry2):
            gbase = g * 16
            dv = jnp.zeros((16,), jnp.float32)
            for l in range(16):
                p = gbase + l
                acc = jnp.zeros((16,), jnp.float32)
                for cc in range(H // 32):
                    prod = (bufa_v[p, pl.ds(cc * 32, 32)]
                            * bufb_v[p, pl.ds(cc * 32, 32)])
                    u0, u1 = plsc.unpack(
                        prod, format=plsc.PackFormat.INTERLEAVED)
                    acc = acc + u0 + u1
                s = jnp.sum(acc)
                dv = jnp.where(lane == l, s, dv)
            dots_v[pl.ds(gbase, 16)] = dv
            return carry2

        lax.fori_loop(0, K // 16, group, 0)

    bufsets = ((ia0, ib0, bufa0, bufb0, si0, sga0, sgb0),
               (ia1, ib1, bufa1, bufb1, si1, sga1, sgb1),
               (ia2, ib2, bufa2, bufb2, si2, sga2, sgb2))
    for s in range(3):
        a_v, b_v, ba, bb, si, sa, sb = bufsets[s]
        cp_idx(s, a_v, b_v, si)
        wait_idx(a_v, b_v, si)
        pltpu.async_copy(xf_hbm.at[a_v], ba, sa)
        pltpu.async_copy(xf_hbm.at[b_v], bb, sb)

    def piter(j, carry):
        more = j + 1 < PCHUNK // 3
        for s in range(3):
            c = 3 * j + s
            a_v, b_v, ba, bb, si, sa, sb = bufsets[s]
            pltpu.make_async_copy(xf_hbm.at[a_v], ba, sa).wait()
            pltpu.make_async_copy(xf_hbm.at[b_v], bb, sb).wait()

            @pl.when(more)
            def _():
                cp_idx(c + 3, a_v, b_v, si)

            dots(ba, bb, dots0)
            pltpu.sync_copy(
                dots0, out_hbm.at[pl.ds(pl.multiple_of(pbase + c * K, 8), K)])

            @pl.when(more)
            def _():
                wait_idx(a_v, b_v, si)
                pltpu.async_copy(xf_hbm.at[a_v], ba, sa)
                pltpu.async_copy(xf_hbm.at[b_v], bb, sb)

        return carry

    lax.fori_loop(0, PCHUNK // 3, piter, 0)


@functools.cache
def _sc_deg():
    return pl.kernel(
        _deg_body,
        mesh=_mesh(),
        compiler_params=pltpu.CompilerParams(
            use_tc_tiling_on_sc=False, needs_layout_passes=False),
        out_type=jax.ShapeDtypeStruct((NC * V * NPAD,), jnp.float32),
        scratch_types=[
            pltpu.VMEM((K,), jnp.int32),
            pltpu.VMEM((K,), jnp.int32),
            pltpu.VMEM((K, 16), jnp.float32),
            pltpu.VMEM((ROWS_PT, 16), jnp.float32),
            pltpu.VMEM((ROWS_PT, 16), jnp.float32),
            pltpu.VMEM((ROWS_PT,), jnp.float32),
            pltpu.VMEM_SHARED((NPAD, 16), jnp.float32),
            pltpu.SemaphoreType.DMA,
            pltpu.SemaphoreType.DMA,
            pltpu.SemaphoreType.DMA,
            pltpu.SemaphoreType.DMA,
        ],
    )


@functools.cache
def _sc_msg():
    return pl.kernel(
        _msg_body,
        mesh=_mesh(),
        compiler_params=pltpu.CompilerParams(use_tc_tiling_on_sc=False),
        out_type=jax.ShapeDtypeStruct((NC, V, NPAD, H), jnp.bfloat16),
        scratch_types=[
            pltpu.VMEM((K,), jnp.int32),
            pltpu.VMEM((K,), jnp.int32),
            pltpu.VMEM((K,), jnp.int32),
            pltpu.VMEM((K,), jnp.int32),
            pltpu.VMEM((K, H), jnp.bfloat16),
            pltpu.VMEM((K, H), jnp.bfloat16),
            pltpu.VMEM((K, H), jnp.bfloat16),
            pltpu.VMEM_SHARED((NPAD, H), jnp.bfloat16),
            pltpu.SemaphoreType.DMA,
            pltpu.SemaphoreType.DMA,
            pltpu.SemaphoreType.DMA,
            pltpu.SemaphoreType.DMA,
            pltpu.SemaphoreType.DMA,
            pltpu.SemaphoreType.DMA,
        ],
    )


@functools.cache
def _sc_link():
    return pl.kernel(
        _link_body,
        mesh=_mesh(),
        compiler_params=pltpu.CompilerParams(
            use_tc_tiling_on_sc=False, needs_layout_passes=False),
        out_type=jax.ShapeDtypeStruct((TP,), jnp.float32),
        scratch_types=(
            [pltpu.VMEM((K,), jnp.int32)] * 6
            + [pltpu.VMEM((K, H), jnp.bfloat16)] * 6
            + [pltpu.VMEM((K,), jnp.float32)]
            + [pltpu.SemaphoreType.DMA] * 9
        ),
    )


_RB = 1000


def _tc_a_body(x_ref, w_ref, cnt_ref, xw_ref, yw_ref, dis_ref):
    xb = x_ref[...]
    for i in range(V):
        c2 = cnt_ref[0, i, :, :1] + cnt_ref[1, i, :, :1]
        dis = lax.rsqrt(c2 + 1.0)
        xw = lax.dot_general(xb, w_ref[i], (((1,), (1,)), ((), ())),
                             preferred_element_type=jnp.float32)
        xw_ref[i] = xw
        yw_ref[i] = (dis * xw).astype(jnp.bfloat16)
        dis_ref[i] = dis


def _tc_a(x, w, cnt):
    return pl.pallas_call(
        _tc_a_body,
        grid=(N // _RB,),
        in_specs=[
            pl.BlockSpec((_RB, H), lambda j: (j, 0)),
            pl.BlockSpec((V, H, H), lambda j: (0, 0, 0)),
            pl.BlockSpec((NC, V, _RB, 1), lambda j: (0, 0, j, 0)),
        ],
        out_specs=[
            pl.BlockSpec((V, _RB, H), lambda j: (0, j, 0)),
            pl.BlockSpec((V, _RB, H), lambda j: (0, j, 0)),
            pl.BlockSpec((V, _RB, 1), lambda j: (0, j, 0)),
        ],
        out_shape=[
            jax.ShapeDtypeStruct((V, N, H), jnp.float32),
            jax.ShapeDtypeStruct((V, N, H), jnp.bfloat16),
            jax.ShapeDtypeStruct((V, N, 1), jnp.float32),
        ],
    )(x, w, cnt)


def _tc_att_body(w_ref, b_ref, bb_ref, att_ref):
    w = w_ref[...]
    u = lax.dot_general(w, b_ref[...], (((2,), (1,)), ((), ())),
                        preferred_element_type=jnp.float32)
    wf = w.reshape(V, H * H)
    uf = u.reshape(V, H * H)
    m = lax.dot_general(wf, uf, (((1,), (1,)), ((), ())),
                        preferred_element_type=jnp.float32)
    m = m + H * bb_ref[0, 0]
    m = m - jnp.max(m, axis=1, keepdims=True)
    e = jnp.exp(m)
    att_ref[...] = e / jnp.sum(e, axis=1, keepdims=True)


def _tc_att(w, b, bb):
    return pl.pallas_call(
        _tc_att_body,
        in_specs=[
            pl.BlockSpec((V, H, H), lambda: (0, 0, 0)),
            pl.BlockSpec((H, H), lambda: (0, 0)),
            pl.BlockSpec(memory_space=pltpu.SMEM),
        ],
        out_specs=pl.BlockSpec((V, V), lambda: (0, 0)),
        out_shape=jax.ShapeDtypeStruct((V, V), jnp.float32),
    )(w, b, bb.reshape(1, 1))


def _tc_c_body(mp_ref, xw_ref, dis_ref, b1_ref, att_ref, wct_ref, bc_ref,
               w2t_ref, xw2_ref, yw2_ref):
    xms = []
    for i in range(V):
        msg = (mp_ref[0, i].astype(jnp.float32)
               + mp_ref[1, i].astype(jnp.float32))
        d = dis_ref[i]
        xm1 = _elu(d * msg + d * d * xw_ref[i] + b1_ref[i:i + 1, :])
        tmp = (att_ref[i, 0] * xw_ref[0] + att_ref[i, 1] * xw_ref[1]
               + att_ref[i, 2] * xw_ref[2])
        xms.append((1.0 - ALPHA) + xm1 + _elu(ALPHA * tmp))
    xcat = jnp.concatenate(xms, axis=1)
    xc = _elu(jnp.dot(xcat, wct_ref[...],
                      preferred_element_type=jnp.float32) + bc_ref[...])
    for i in range(V):
        xw2 = jnp.dot(xc, w2t_ref[i], preferred_element_type=jnp.float32)
        xw2_ref[i] = xw2
        yw2_ref[i] = (dis_ref[i] * xw2).astype(jnp.bfloat16)


def _tc_c(mp1, xw1, dis, b1, att1, wct, bc, w2t):
    return pl.pallas_call(
        _tc_c_body,
        grid=(N // _RB,),
        in_specs=[
            pl.BlockSpec((NC, V, _RB, H), lambda j: (0, 0, j, 0)),
            pl.BlockSpec((V, _RB, H), lambda j: (0, j, 0)),
            pl.BlockSpec((V, _RB, 1), lambda j: (0, j, 0)),
            pl.BlockSpec((V, H), lambda j: (0, 0)),
            pl.BlockSpec(memory_space=pltpu.SMEM),
            pl.BlockSpec((V * H, H), lambda j: (0, 0)),
            pl.BlockSpec((1, H), lambda j: (0, 0)),
            pl.BlockSpec((V, H, H), lambda j: (0, 0, 0)),
        ],
        out_specs=[
            pl.BlockSpec((V, _RB, H), lambda j: (0, j, 0)),
            pl.BlockSpec((V, _RB, H), lambda j: (0, j, 0)),
        ],
        out_shape=[
            jax.ShapeDtypeStruct((V, N, H), jnp.float32),
            jax.ShapeDtypeStruct((V, N, H), jnp.bfloat16),
        ],
    )(mp1, xw1, dis, b1, att1, wct, bc, w2t)


def _tc_d_body(mp_ref, xw_ref, dis_ref, b2_ref, att_ref, xf_ref):
    for i in range(V):
        msg = (mp_ref[0, i].astype(jnp.float32)
               + mp_ref[1, i].astype(jnp.float32))
        d = dis_ref[i]
        xm2 = d * msg + d * d * xw_ref[i] + b2_ref[i:i + 1, :]
        tmp = (att_ref[i, 0] * xw_ref[0] + att_ref[i, 1] * xw_ref[1]
               + att_ref[i, 2] * xw_ref[2])
        xf_ref[i] = ((1.0 - ALPHA) + xm2 + _elu(ALPHA * tmp)).astype(jnp.bfloat16)


def _tc_d(mp2, xw2, dis, b2, att2):
    return pl.pallas_call(
        _tc_d_body,
        grid=(N // _RB,),
        in_specs=[
            pl.BlockSpec((NC, V, _RB, H), lambda j: (0, 0, j, 0)),
            pl.BlockSpec((V, _RB, H), lambda j: (0, j, 0)),
            pl.BlockSpec((V, _RB, 1), lambda j: (0, j, 0)),
            pl.BlockSpec((V, H), lambda j: (0, 0)),
            pl.BlockSpec(memory_space=pltpu.SMEM),
        ],
        out_specs=pl.BlockSpec((V, _RB, H), lambda j: (0, j, 0)),
        out_shape=jax.ShapeDtypeStruct((V, N, H), jnp.bfloat16),
    )(mp2, xw2, dis, b2, att2)


def kernel(x, edge_index, edges, edges_neg, W1, b1, W2, b2, B1, bb1, B2, bb2,
           Wc, bc):
    edge_index = edge_index.astype(jnp.int32)
    offs = (jnp.arange(V, dtype=jnp.int32) * N)[:, None]
    src_flat = (edge_index[:, 0, :] + offs).reshape(-1)
    dst_flat = edge_index[:, 1, :].reshape(-1)
    pcnt = _sc_deg()(dst_flat).reshape(NC, V, NPAD, 1)
    xw1, yw1, dis = _tc_a(x, W1, pcnt)

    att1 = _tc_att(W1, B1, bb1)
    att2 = _tc_att(W2, B2, bb2)

    mp1 = _sc_msg()(src_flat, dst_flat, yw1.reshape(V * N, H))
    xw2, yw2 = _tc_c(mp1, xw1, dis, b1, att1,
                     Wc.T, bc.reshape(1, H), jnp.transpose(W2, (0, 2, 1)))

    mp2 = _sc_msg()(src_flat, dst_flat, yw2.reshape(V * N, H))
    xf = _tc_d(mp2, xw2, dis, b2, att2)

    xff = xf.reshape(V * N, H)
    ia = jnp.concatenate(
        [edges[:, :, 0].astype(jnp.int32) + offs,
         edges_neg[:, :, 0].astype(jnp.int32) + offs], axis=1).reshape(-1)
    ib = jnp.concatenate(
        [edges[:, :, 1].astype(jnp.int32) + offs,
         edges_neg[:, :, 1].astype(jnp.int32) + offs], axis=1).reshape(-1)
    ia = jnp.concatenate([ia, jnp.zeros((TP - V * 2 * P,), jnp.int32)])
    ib = jnp.concatenate([ib, jnp.zeros((TP - V * 2 * P,), jnp.int32)])

    scores = _sc_link()(xff, ia, ib)
    return scores[:V * 2 * P].reshape(V, 2 * P)


# speedup vs baseline: 1.1417x; 1.1417x over previous
import functools

import jax
import jax.numpy as jnp
from jax import lax
from jax.experimental import pallas as pl
from jax.experimental.pallas import tpu as pltpu
from jax.experimental.pallas import tpu_sc as plsc

N = 10000
H = 128
V = 3
E = 320000
P = 50000
ALPHA = 0.5

NPAD = 10240
NC = 2
NS = 16
NW = NC * NS
K = 80
EPADV = E
EPT = EPADV // NW
NCHUNK = EPT // K
ROWS_PT = NPAD // NS
TP = 307200
PPT = TP // NW
PCHUNK = PPT // K

def _elu(x):
    return jnp.where(x > 0, x, jnp.exp(jnp.minimum(x, 0.0)) - 1.0)


def _mesh():
    return plsc.VectorSubcoreMesh(
        core_axis_name="c", subcore_axis_name="s",
        num_cores=NC, num_subcores=NS)


def _deg_body(dst_hbm, out_hbm, idx0_v, idx1_v, ones_v, zeros_v, stg_v, cmp_v,
              acc_sh, si0, si1, ss0, ss1):
    cid = lax.axis_index("c")
    sid = lax.axis_index("s")
    wid = cid * NS + sid
    ebase = wid * EPT
    rbase = sid * ROWS_PT

    def init_ones(r, carry):
        ones_v[r, :] = jnp.ones((16,), jnp.float32)
        return carry

    lax.fori_loop(0, K, init_ones, 0)

    def init_zeros(r, carry):
        zeros_v[r, :] = jnp.zeros((16,), jnp.float32)
        return carry

    lax.fori_loop(0, ROWS_PT, init_zeros, 0)

    for i in range(V):
        pltpu.sync_copy(zeros_v, acc_sh.at[pl.ds(rbase, ROWS_PT)])
        plsc.subcore_barrier()
        base = i * EPADV + ebase

        pltpu.async_copy(dst_hbm.at[pl.ds(pl.multiple_of(base, 8), K)],
                         idx0_v, si0)

        def piter(g, carry):
            c0 = 2 * g

            @pl.when(g > 0)
            def _():
                pltpu.make_async_copy(ones_v, acc_sh.at[idx1_v], ss1).wait()
            d_i1 = pltpu.async_copy(
                dst_hbm.at[pl.ds(pl.multiple_of(base + (c0 + 1) * K, 8), K)],
                idx1_v, si1)
            pltpu.make_async_copy(dst_hbm, idx0_v, si0).wait()
            d_s0 = pltpu.async_copy(ones_v, acc_sh.at[idx0_v], ss0, add=True)
            d_i1.wait()
            pltpu.async_copy(ones_v, acc_sh.at[idx1_v], ss1, add=True)
            d_s0.wait()
            pltpu.async_copy(
                dst_hbm.at[pl.ds(pl.multiple_of(base + (c0 + 2) * K, 8), K)],
                idx0_v, si0)
            return carry

        lax.fori_loop(0, (NCHUNK - 1) // 2, piter, 0)
        pltpu.make_async_copy(ones_v, acc_sh.at[idx1_v], ss1).wait()
        pltpu.make_async_copy(dst_hbm, idx0_v, si0).wait()
        pltpu.async_copy(ones_v, acc_sh.at[idx0_v], ss0, add=True)
        pltpu.make_async_copy(ones_v, acc_sh.at[idx0_v], ss0).wait()

        plsc.subcore_barrier()
        pltpu.sync_copy(acc_sh.at[pl.ds(rbase, ROWS_PT)], stg_v)
        czero = jnp.zeros((16,), jnp.int32)
        lane16 = lax.iota(jnp.int32, 16)

        def compact(r, carry):
            cnt16 = plsc.load_gather(stg_v, [r * 16 + lane16, czero])
            cmp_v[pl.ds(r * 16, 16)] = cnt16
            return carry

        lax.fori_loop(0, ROWS_PT // 16, compact, 0)
        pltpu.sync_copy(
            cmp_v, out_hbm.at[pl.ds((cid * V + i) * NPAD + rbase, ROWS_PT)])
        plsc.subcore_barrier()


def _msg_body(src_hbm, dst_hbm, yw_hbm, out_hbm, gidx0, didx0, gidx1, didx1,
              gidx2, didx2, gidx3, didx3, rows0, rows1, rows2, rows3, zrows_v,
              acc_sh, si0, si1, si2, si3, sg0, sg1, sg2, sg3, ss0, ss1, ss2,
              ss3):
    cid = lax.axis_index("c")
    sid = lax.axis_index("s")
    wid = cid * NS + sid
    rbase = sid * ROWS_PT
    bufsets = ((gidx0, didx0, rows0, si0, sg0, ss0),
               (gidx1, didx1, rows1, si1, sg1, ss1),
               (gidx2, didx2, rows2, si2, sg2, ss2),
               (gidx3, didx3, rows3, si3, sg3, ss3))

    def init_zrows(r, carry):
        for cc in range(H // 32):
            zrows_v[r, pl.ds(cc * 32, 32)] = jnp.zeros((32,), jnp.bfloat16)
        return carry

    lax.fori_loop(0, K, init_zrows, 0)

    for i in range(V):
        base = i * EPADV + wid * EPT

        def cp_idx(c, g_v, d_v, sem):
            off = pl.multiple_of(base + c * K, 8)
            pltpu.async_copy(src_hbm.at[pl.ds(off, K)], g_v, sem)
            pltpu.async_copy(dst_hbm.at[pl.ds(off, K)], d_v, sem)

        def wait_idx(g_v, d_v, sem):
            pltpu.make_async_copy(src_hbm, g_v, sem).wait()
            pltpu.make_async_copy(dst_hbm, d_v, sem).wait()

        for kk in range(ROWS_PT // K):
            pltpu.sync_copy(zrows_v, acc_sh.at[pl.ds(rbase + kk * K, K)])
        plsc.subcore_barrier()

        for s in range(3):
            g_v, d_v, r_v, si, sg, ss = bufsets[s]
            cp_idx(s, g_v, d_v, si)
            wait_idx(g_v, d_v, si)
            pltpu.async_copy(yw_hbm.at[g_v], r_v, sg)

        def piter(j, carry):
            for s in range(4):
                c = 4 * j + s
                g_v, d_v, r_v, si, sg, ss = bufsets[s]
                gq_v, dq_v, rq_v, siq, sgq, ssq = bufsets[(s + 3) % 4]
                pltpu.make_async_copy(yw_hbm.at[g_v], r_v, sg).wait()
                pltpu.async_copy(r_v, acc_sh.at[d_v], ss, add=True)

                def refill():
                    pltpu.make_async_copy(rq_v, acc_sh.at[dq_v], ssq).wait()
                    cp_idx(c + 3, gq_v, dq_v, siq)
                    wait_idx(gq_v, dq_v, siq)
                    pltpu.async_copy(yw_hbm.at[gq_v], rq_v, sgq)

                if s == 0:
                    @pl.when(j > 0)
                    def _():
                        refill()

                    @pl.when(j == 0)
                    def _():
                        cp_idx(3, gq_v, dq_v, siq)
                        wait_idx(gq_v, dq_v, siq)
                        pltpu.async_copy(yw_hbm.at[gq_v], rq_v, sgq)
                elif s == 1:
                    refill()
                else:
                    @pl.when(j < (NCHUNK - 1) // 4 - 1)
                    def _():
                        refill()
            return carry

        lax.fori_loop(0, (NCHUNK - 1) // 4, piter, 0)
        pltpu.make_async_copy(rows1, acc_sh.at[didx1], ss1).wait()
        pltpu.make_async_copy(rows2, acc_sh.at[didx2], ss2).wait()
        pltpu.make_async_copy(rows3, acc_sh.at[didx3], ss3).wait()
        g_v, d_v, r_v, si, sg, ss = bufsets[0]
        pltpu.make_async_copy(yw_hbm.at[g_v], r_v, sg).wait()
        pltpu.async_copy(r_v, acc_sh.at[d_v], ss, add=True)
        pltpu.make_async_copy(r_v, acc_sh.at[d_v], ss).wait()

        plsc.subcore_barrier()
        for kk in range(ROWS_PT // K):
            pltpu.sync_copy(acc_sh.at[pl.ds(rbase + kk * K, K)], rows0)
            pltpu.sync_copy(rows0, out_hbm.at[cid, i, pl.ds(rbase + kk * K, K)])
        plsc.subcore_barrier()


def _link_body(xf_hbm, ia_hbm, ib_hbm, out_hbm, ia0, ib0, ia1, ib1, ia2, ib2,
               bufa0, bufb0, bufa1, bufb1, bufa2, bufb2, dots0, si0, si1, si2,
               sga0, sgb0, sga1, sgb1, sga2, sgb2):
    cid = lax.axis_index("c")
    sid = lax.axis_index("s")
    wid = cid * NS + sid
    pbase = wid * PPT
    lane = lax.iota(jnp.int32, 16)

    def cp_idx(c, a_v, b_v, sem):
        off = pl.multiple_of(pbase + c * K, 8)
        pltpu.async_copy(ia_hbm.at[pl.ds(off, K)], a_v, sem)
        pltpu.async_copy(ib_hbm.at[pl.ds(off, K)], b_v, sem)

    def wait_idx(a_v, b_v, sem):
        pltpu.make_async_copy(ia_hbm, a_v, sem).wait()
        pltpu.make_async_copy(ib_hbm, b_v, sem).wait()

    def dots(bufa_v, bufb_v, dots_v):
        def group(g, carry2):
            gbase = g * 16
            dv = jnp.zeros((16,), jnp.float32)
            for l in range(16):
                p = gbase + l
                acc = jnp.zeros((16,), jnp.float32)
                for cc in range(H // 32):
                    prod = (bufa_v[p, pl.ds(cc * 32, 32)]
                            * bufb_v[p, pl.ds(cc * 32, 32)])
                    u0, u1 = plsc.unpack(
                        prod, format=plsc.PackFormat.INTERLEAVED)
                    acc = acc + u0 + u1
                s = jnp.sum(acc)
                dv = jnp.where(lane == l, s, dv)
            dots_v[pl.ds(gbase, 16)] = dv
            return carry2

        lax.fori_loop(0, K // 16, group, 0)

    bufsets = ((ia0, ib0, bufa0, bufb0, si0, sga0, sgb0),
               (ia1, ib1, bufa1, bufb1, si1, sga1, sgb1),
               (ia2, ib2, bufa2, bufb2, si2, sga2, sgb2))
    for s in range(3):
        a_v, b_v, ba, bb, si, sa, sb = bufsets[s]
        cp_idx(s, a_v, b_v, si)
        wait_idx(a_v, b_v, si)
        pltpu.async_copy(xf_hbm.at[a_v], ba, sa)
        pltpu.async_copy(xf_hbm.at[b_v], bb, sb)

    def piter(j, carry):
        more = j + 1 < PCHUNK // 3
        for s in range(3):
            c = 3 * j + s
            a_v, b_v, ba, bb, si, sa, sb = bufsets[s]
            pltpu.make_async_copy(xf_hbm.at[a_v], ba, sa).wait()
            pltpu.make_async_copy(xf_hbm.at[b_v], bb, sb).wait()

            @pl.when(more)
            def _():
                cp_idx(c + 3, a_v, b_v, si)

            dots(ba, bb, dots0)
            pltpu.sync_copy(
                dots0, out_hbm.at[pl.ds(pl.multiple_of(pbase + c * K, 8), K)])

            @pl.when(more)
            def _():
                wait_idx(a_v, b_v, si)
                pltpu.async_copy(xf_hbm.at[a_v], ba, sa)
                pltpu.async_copy(xf_hbm.at[b_v], bb, sb)

        return carry

    lax.fori_loop(0, PCHUNK // 3, piter, 0)


@functools.cache
def _sc_deg():
    return pl.kernel(
        _deg_body,
        mesh=_mesh(),
        compiler_params=pltpu.CompilerParams(
            use_tc_tiling_on_sc=False, needs_layout_passes=False),
        out_type=jax.ShapeDtypeStruct((NC * V * NPAD,), jnp.float32),
        scratch_types=[
            pltpu.VMEM((K,), jnp.int32),
            pltpu.VMEM((K,), jnp.int32),
            pltpu.VMEM((K, 16), jnp.float32),
            pltpu.VMEM((ROWS_PT, 16), jnp.float32),
            pltpu.VMEM((ROWS_PT, 16), jnp.float32),
            pltpu.VMEM((ROWS_PT,), jnp.float32),
            pltpu.VMEM_SHARED((NPAD, 16), jnp.float32),
            pltpu.SemaphoreType.DMA,
            pltpu.SemaphoreType.DMA,
            pltpu.SemaphoreType.DMA,
            pltpu.SemaphoreType.DMA,
        ],
    )


@functools.cache
def _sc_msg():
    return pl.kernel(
        _msg_body,
        mesh=_mesh(),
        compiler_params=pltpu.CompilerParams(use_tc_tiling_on_sc=False),
        out_type=jax.ShapeDtypeStruct((NC, V, NPAD, H), jnp.bfloat16),
        scratch_types=(
            [pltpu.VMEM((K,), jnp.int32)] * 8
            + [pltpu.VMEM((K, H), jnp.bfloat16)] * 5
            + [pltpu.VMEM_SHARED((NPAD, H), jnp.bfloat16)]
            + [pltpu.SemaphoreType.DMA] * 12
        ),
    )


@functools.cache
def _sc_link():
    return pl.kernel(
        _link_body,
        mesh=_mesh(),
        compiler_params=pltpu.CompilerParams(
            use_tc_tiling_on_sc=False, needs_layout_passes=False),
        out_type=jax.ShapeDtypeStruct((TP,), jnp.float32),
        scratch_types=(
            [pltpu.VMEM((K,), jnp.int32)] * 6
            + [pltpu.VMEM((K, H), jnp.bfloat16)] * 6
            + [pltpu.VMEM((K,), jnp.float32)]
            + [pltpu.SemaphoreType.DMA] * 9
        ),
    )


_RB = 1000


def _tc_a_body(x_ref, w_ref, cnt_ref, xw_ref, yw_ref, dis_ref):
    xb = x_ref[...]
    for i in range(V):
        c2 = cnt_ref[0, i, :, :1] + cnt_ref[1, i, :, :1]
        dis = lax.rsqrt(c2 + 1.0)
        xw = lax.dot_general(xb, w_ref[i], (((1,), (1,)), ((), ())),
                             preferred_element_type=jnp.float32)
        xw_ref[i] = xw
        yw_ref[i] = (dis * xw).astype(jnp.bfloat16)
        dis_ref[i] = dis


def _tc_a(x, w, cnt):
    return pl.pallas_call(
        _tc_a_body,
        grid=(N // _RB,),
        in_specs=[
            pl.BlockSpec((_RB, H), lambda j: (j, 0)),
            pl.BlockSpec((V, H, H), lambda j: (0, 0, 0)),
            pl.BlockSpec((NC, V, _RB, 1), lambda j: (0, 0, j, 0)),
        ],
        out_specs=[
            pl.BlockSpec((V, _RB, H), lambda j: (0, j, 0)),
            pl.BlockSpec((V, _RB, H), lambda j: (0, j, 0)),
            pl.BlockSpec((V, _RB, 1), lambda j: (0, j, 0)),
        ],
        out_shape=[
            jax.ShapeDtypeStruct((V, N, H), jnp.float32),
            jax.ShapeDtypeStruct((V, N, H), jnp.bfloat16),
            jax.ShapeDtypeStruct((V, N, 1), jnp.float32),
        ],
    )(x, w, cnt)


def _tc_att_body(w_ref, b_ref, bb_ref, att_ref):
    w = w_ref[...]
    u = lax.dot_general(w, b_ref[...], (((2,), (1,)), ((), ())),
                        preferred_element_type=jnp.float32)
    wf = w.reshape(V, H * H)
    uf = u.reshape(V, H * H)
    m = lax.dot_general(wf, uf, (((1,), (1,)), ((), ())),
                        preferred_element_type=jnp.float32)
    m = m + H * bb_ref[0, 0]
    m = m - jnp.max(m, axis=1, keepdims=True)
    e = jnp.exp(m)
    att_ref[...] = e / jnp.sum(e, axis=1, keepdims=True)


def _tc_att(w, b, bb):
    return pl.pallas_call(
        _tc_att_body,
        in_specs=[
            pl.BlockSpec((V, H, H), lambda: (0, 0, 0)),
            pl.BlockSpec((H, H), lambda: (0, 0)),
            pl.BlockSpec(memory_space=pltpu.SMEM),
        ],
        out_specs=pl.BlockSpec((V, V), lambda: (0, 0)),
        out_shape=jax.ShapeDtypeStruct((V, V), jnp.float32),
    )(w, b, bb.reshape(1, 1))


def _tc_c_body(mp_ref, xw_ref, dis_ref, b1_ref, att_ref, wct_ref, bc_ref,
               w2t_ref, xw2_ref, yw2_ref):
    xms = []
    for i in range(V):
        msg = (mp_ref[0, i].astype(jnp.float32)
               + mp_ref[1, i].astype(jnp.float32))
        d = dis_ref[i]
        xm1 = _elu(d * msg + d * d * xw_ref[i] + b1_ref[i:i + 1, :])
        tmp = (att_ref[i, 0] * xw_ref[0] + att_ref[i, 1] * xw_ref[1]
               + att_ref[i, 2] * xw_ref[2])
        xms.append((1.0 - ALPHA) + xm1 + _elu(ALPHA * tmp))
    xcat = jnp.concatenate(xms, axis=1)
    xc = _elu(jnp.dot(xcat, wct_ref[...],
                      preferred_element_type=jnp.float32) + bc_ref[...])
    for i in range(V):
        xw2 = jnp.dot(xc, w2t_ref[i], preferred_element_type=jnp.float32)
        xw2_ref[i] = xw2
        yw2_ref[i] = (dis_ref[i] * xw2).astype(jnp.bfloat16)


def _tc_c(mp1, xw1, dis, b1, att1, wct, bc, w2t):
    return pl.pallas_call(
        _tc_c_body,
        grid=(N // _RB,),
        in_specs=[
            pl.BlockSpec((NC, V, _RB, H), lambda j: (0, 0, j, 0)),
            pl.BlockSpec((V, _RB, H), lambda j: (0, j, 0)),
            pl.BlockSpec((V, _RB, 1), lambda j: (0, j, 0)),
            pl.BlockSpec((V, H), lambda j: (0, 0)),
            pl.BlockSpec(memory_space=pltpu.SMEM),
            pl.BlockSpec((V * H, H), lambda j: (0, 0)),
            pl.BlockSpec((1, H), lambda j: (0, 0)),
            pl.BlockSpec((V, H, H), lambda j: (0, 0, 0)),
        ],
        out_specs=[
            pl.BlockSpec((V, _RB, H), lambda j: (0, j, 0)),
            pl.BlockSpec((V, _RB, H), lambda j: (0, j, 0)),
        ],
        out_shape=[
            jax.ShapeDtypeStruct((V, N, H), jnp.float32),
            jax.ShapeDtypeStruct((V, N, H), jnp.bfloat16),
        ],
    )(mp1, xw1, dis, b1, att1, wct, bc, w2t)


def _tc_d_body(mp_ref, xw_ref, dis_ref, b2_ref, att_ref, xf_ref):
    for i in range(V):
        msg = (mp_ref[0, i].astype(jnp.float32)
               + mp_ref[1, i].astype(jnp.float32))
        d = dis_ref[i]
        xm2 = d * msg + d * d * xw_ref[i] + b2_ref[i:i + 1, :]
        tmp = (att_ref[i, 0] * xw_ref[0] + att_ref[i, 1] * xw_ref[1]
               + att_ref[i, 2] * xw_ref[2])
        xf_ref[i] = ((1.0 - ALPHA) + xm2 + _elu(ALPHA * tmp)).astype(jnp.bfloat16)


def _tc_d(mp2, xw2, dis, b2, att2):
    return pl.pallas_call(
        _tc_d_body,
        grid=(N // _RB,),
        in_specs=[
            pl.BlockSpec((NC, V, _RB, H), lambda j: (0, 0, j, 0)),
            pl.BlockSpec((V, _RB, H), lambda j: (0, j, 0)),
            pl.BlockSpec((V, _RB, 1), lambda j: (0, j, 0)),
            pl.BlockSpec((V, H), lambda j: (0, 0)),
            pl.BlockSpec(memory_space=pltpu.SMEM),
        ],
        out_specs=pl.BlockSpec((V, _RB, H), lambda j: (0, j, 0)),
        out_shape=jax.ShapeDtypeStruct((V, N, H), jnp.bfloat16),
    )(mp2, xw2, dis, b2, att2)


def kernel(x, edge_index, edges, edges_neg, W1, b1, W2, b2, B1, bb1, B2, bb2,
           Wc, bc):
    edge_index = edge_index.astype(jnp.int32)
    offs = (jnp.arange(V, dtype=jnp.int32) * N)[:, None]
    src_flat = (edge_index[:, 0, :] + offs).reshape(-1)
    dst_flat = edge_index[:, 1, :].reshape(-1)
    pcnt = _sc_deg()(dst_flat).reshape(NC, V, NPAD, 1)
    xw1, yw1, dis = _tc_a(x, W1, pcnt)

    att1 = _tc_att(W1, B1, bb1)
    att2 = _tc_att(W2, B2, bb2)

    mp1 = _sc_msg()(src_flat, dst_flat, yw1.reshape(V * N, H))
    xw2, yw2 = _tc_c(mp1, xw1, dis, b1, att1,
                     Wc.T, bc.reshape(1, H), jnp.transpose(W2, (0, 2, 1)))

    mp2 = _sc_msg()(src_flat, dst_flat, yw2.reshape(V * N, H))
    xf = _tc_d(mp2, xw2, dis, b2, att2)

    xff = xf.reshape(V * N, H)
    ia = jnp.concatenate(
        [edges[:, :, 0].astype(jnp.int32) + offs,
         edges_neg[:, :, 0].astype(jnp.int32) + offs], axis=1).reshape(-1)
    ib = jnp.concatenate(
        [edges[:, :, 1].astype(jnp.int32) + offs,
         edges_neg[:, :, 1].astype(jnp.int32) + offs], axis=1).reshape(-1)
    ia = jnp.concatenate([ia, jnp.zeros((TP - V * 2 * P,), jnp.int32)])
    ib = jnp.concatenate([ib, jnp.zeros((TP - V * 2 * P,), jnp.int32)])

    scores = _sc_link()(xff, ia, ib)
    return scores[:V * 2 * P].reshape(V, 2 * P)
